# Initial kernel scaffold; baseline (speedup 1.0000x reference)
import jax
import jax.numpy as jnp
from jax.experimental import pallas as pl


def kernel(hidden_states, cos, sin, wq, wk, ww):
    raise NotImplementedError("write your pallas kernel here")


# speedup vs baseline: 1.0473x; 1.0473x over previous
import functools

import jax
import jax.numpy as jnp
import numpy as np
from jax.experimental import pallas as pl
from jax.experimental.pallas import tpu as pltpu

B, S, DM = 1, 2048, 2048
H, D, RD, TOPK = 12, 64, 32, 512
BQ = 256


def _swap_pairs(x):
    ncols = x.shape[-1]
    col = jax.lax.broadcasted_iota(jnp.int32, x.shape, x.ndim - 1)
    fwd = pltpu.roll(x, ncols - 1, axis=x.ndim - 1)
    bwd = pltpu.roll(x, 1, axis=x.ndim - 1)
    return jnp.where(col % 2 == 0, fwd, bwd)


def _stage_a(hid_ref, wq_ref, wk_ref, ww_ref, cq_ref, sq_ref, ck_ref, sk_ref,
             q_ref, k_ref, w_ref):
    h = hid_ref[...]
    q = jnp.dot(h, wq_ref[...], preferred_element_type=jnp.float32)
    q_ref[...] = q * cq_ref[...] + _swap_pairs(q) * sq_ref[...]
    k = jnp.dot(h, wk_ref[...], preferred_element_type=jnp.float32)
    k_ref[...] = k * ck_ref[...] + _swap_pairs(k) * sk_ref[...]
    wl = jnp.dot(h, ww_ref[...], preferred_element_type=jnp.float32)
    wl = wl - jnp.max(wl, axis=-1, keepdims=True)
    e = jnp.exp(wl)
    w_ref[...] = e / jnp.sum(e, axis=-1, keepdims=True) * (D ** -0.5)


def _stage_b(q_ref, kt_ref, w_ref, s_ref):
    q = q_ref[...]
    w = w_ref[...]
    kt = kt_ref[...]
    acc = jnp.zeros((BQ, S), jnp.float32)
    for h in range(H):
        lg = jnp.dot(q[:, h * D:(h + 1) * D], kt,
                     preferred_element_type=jnp.float32)
        acc = acc + jnp.maximum(lg, 0.0) * w[:, h:h + 1]
    row = pl.program_id(0) * BQ + jax.lax.broadcasted_iota(jnp.int32, (BQ, S), 0)
    col = jax.lax.broadcasted_iota(jnp.int32, (BQ, S), 1)
    s_ref[...] = jnp.where(col <= row, acc, -col.astype(jnp.float32))


def _rope_tables(cos, sin):
    half = RD // 2
    c = cos[:, :half]
    s = sin[:, :half]
    cosF = jnp.repeat(c, 2, axis=1)
    sinF = jnp.stack([-s, s], axis=-1).reshape(S, RD)
    ones = jnp.ones((S, D - RD), jnp.float32)
    zeros = jnp.zeros((S, D - RD), jnp.float32)
    cos64 = jnp.concatenate([ones, cosF], axis=1)
    sin64 = jnp.concatenate([zeros, sinF], axis=1)
    cosQ = jnp.tile(cos64, (1, H))
    sinQ = jnp.tile(sin64, (1, H))
    return cosQ, sinQ, cos64, sin64


@jax.jit
def kernel(hidden_states, cos, sin, wq, wk, ww):
    hid = hidden_states[0]
    cosQ, sinQ, cosK, sinK = _rope_tables(cos[0], sin[0])

    nblk = S // BQ
    q_rope, k_rope, w = pl.pallas_call(
        _stage_a,
        grid=(nblk,),
        in_specs=[
            pl.BlockSpec((BQ, DM), lambda i: (i, 0)),
            pl.BlockSpec((DM, H * D), lambda i: (0, 0)),
            pl.BlockSpec((DM, D), lambda i: (0, 0)),
            pl.BlockSpec((DM, H), lambda i: (0, 0)),
            pl.BlockSpec((BQ, H * D), lambda i: (i, 0)),
            pl.BlockSpec((BQ, H * D), lambda i: (i, 0)),
            pl.BlockSpec((BQ, D), lambda i: (i, 0)),
            pl.BlockSpec((BQ, D), lambda i: (i, 0)),
        ],
        out_specs=[
            pl.BlockSpec((BQ, H * D), lambda i: (i, 0)),
            pl.BlockSpec((BQ, D), lambda i: (i, 0)),
            pl.BlockSpec((BQ, H), lambda i: (i, 0)),
        ],
        out_shape=[
            jax.ShapeDtypeStruct((S, H * D), jnp.float32),
            jax.ShapeDtypeStruct((S, D), jnp.float32),
            jax.ShapeDtypeStruct((S, H), jnp.float32),
        ],
    )(hid, wq, wk, ww, cosQ, sinQ, cosK, sinK)

    kt = k_rope.T

    scores = pl.pallas_call(
        _stage_b,
        grid=(nblk,),
        in_specs=[
            pl.BlockSpec((BQ, H * D), lambda i: (i, 0)),
            pl.BlockSpec((D, S), lambda i: (0, 0)),
            pl.BlockSpec((BQ, H), lambda i: (i, 0)),
        ],
        out_specs=pl.BlockSpec((BQ, S), lambda i: (i, 0)),
        out_shape=jax.ShapeDtypeStruct((S, S), jnp.float32),
    )(q_rope, kt, w)

    tv, ti = jax.lax.top_k(scores, TOPK)
    tv = jnp.where(tv < 0.0, jnp.float32(-1e9), tv)
    return tv[None], ti[None]
